# baseline (speedup 1.0000x reference)
import functools

import jax
import jax.numpy as jnp
from jax import lax
from jax.experimental import pallas as pl
from jax.experimental.pallas import tpu as pltpu
from jax.experimental.pallas import tpu_sc as plsc

NUM_SPK = 100000
EMBD_DIM = 64
BATCH = 16384

BS = 512
NBLK = (NUM_SPK + BS - 1) // BS
PACKED_ROWS = NBLK * (BS // 2)

NUM_CORES = 2
NUM_SUBCORES = 16
NW = NUM_CORES * NUM_SUBCORES
B_PER_W = BATCH // NW
IDX_CHUNK = 128
N_CHUNKS = B_PER_W // IDX_CHUNK
N_VEC = B_PER_W // 16


def _relayout_block(tt_ref, o_ref):
    a = tt_ref[...].T
    o_ref[...] = jnp.concatenate([a[: BS // 2], a[BS // 2:]], axis=1)


def _tc_relayout(tt):
    return pl.pallas_call(
        _relayout_block,
        grid=(NBLK,),
        in_specs=[pl.BlockSpec((EMBD_DIM, BS), lambda r: (0, r))],
        out_specs=pl.BlockSpec((BS // 2, 128), lambda r: (r, 0)),
        out_shape=jax.ShapeDtypeStruct((PACKED_ROWS, 128), jnp.float32),
    )(tt)


_mesh = plsc.VectorSubcoreMesh(core_axis_name="c", subcore_axis_name="s")


@functools.partial(
    pl.kernel,
    mesh=_mesh,
    compiler_params=pltpu.CompilerParams(
        use_tc_tiling_on_sc=True, needs_layout_passes=False
    ),
    out_type=jax.ShapeDtypeStruct((EMBD_DIM, BATCH), jnp.float32),
    scratch_types=[
        pltpu.VMEM((N_CHUNKS, IDX_CHUNK), jnp.int32),
        pltpu.VMEM((N_CHUNKS, IDX_CHUNK), jnp.int32),
        pltpu.VMEM((N_CHUNKS, IDX_CHUNK), jnp.int32),
        pltpu.VMEM((B_PER_W, 128), jnp.float32),
        pltpu.VMEM((8, 8, B_PER_W), jnp.float32),
        pltpu.SemaphoreType.DMA,
    ],
)
def _sc_gather(t128_hbm, idx_hbm, out_hbm, raw_v, row_v, off_v, rows_v,
               tb_v, sem):
    wid = lax.axis_index("s") * NUM_CORES + lax.axis_index("c")
    base = wid * B_PER_W
    pltpu.sync_copy(idx_hbm.at[pl.ds(wid * N_CHUNKS, N_CHUNKS)], raw_v)
    for t in range(N_VEC):
        c, s = t // 8, (t % 8) * 16
        i = raw_v[c, pl.ds(s, 16)]
        p = lax.bitwise_and(i, 511)
        row = lax.add(
            lax.shift_left(lax.shift_right_logical(i, 9), 8),
            lax.bitwise_and(p, 255),
        )
        off = lax.shift_left(lax.shift_right_logical(p, 8), 6)
        row_v[c, pl.ds(s, 16)] = row
        off_v[c, pl.ds(s, 16)] = off
    copies = []
    for j in range(N_CHUNKS):
        copies.append(
            pltpu.async_copy(
                t128_hbm.at[row_v.at[j]],
                rows_v.at[pl.ds(j * IDX_CHUNK, IDX_CHUNK)],
                sem,
            )
        )
    for c in copies:
        c.wait()
    lanes = lax.iota(jnp.int32, 16)
    for t in range(N_VEC):
        c, s = t // 8, (t % 8) * 16
        b0 = t * 16
        bvec = lax.add(lanes, jnp.int32(b0))
        col0 = off_v[c, pl.ds(s, 16)]
        for j in range(EMBD_DIM):
            col = lax.add(col0, jnp.int32(j))
            val = plsc.load_gather(rows_v, [bvec, col])
            tb_v[j // 8, j % 8, pl.ds(b0, 16)] = val
    for jh in range(8):
        pltpu.sync_copy(
            tb_v.at[jh],
            out_hbm.at[pl.ds(8 * jh, 8), pl.ds(base, B_PER_W)],
        )


def kernel(spk_inds, embedding_table):
    tt = embedding_table.T
    t128 = _tc_relayout(tt)
    idx2d = spk_inds.astype(jnp.int32).reshape(NW * N_CHUNKS, IDX_CHUNK)
    out_t = _sc_gather(t128, idx2d)
    return out_t.T


# speedup vs baseline: 2.0166x; 2.0166x over previous
import functools

import jax
import jax.numpy as jnp
from jax import lax
from jax.experimental import pallas as pl
from jax.experimental.pallas import tpu as pltpu
from jax.experimental.pallas import tpu_sc as plsc

NUM_SPK = 100000
EMBD_DIM = 64
BATCH = 16384

NUM_CORES = 2
NUM_SUBCORES = 16
NW = NUM_CORES * NUM_SUBCORES
B_PER_W = BATCH // NW
IDX_CHUNK = 128
N_CHUNKS = B_PER_W // IDX_CHUNK

_mesh = plsc.VectorSubcoreMesh(core_axis_name="c", subcore_axis_name="s")


@functools.partial(
    pl.kernel,
    mesh=_mesh,
    compiler_params=pltpu.CompilerParams(
        use_tc_tiling_on_sc=True, needs_layout_passes=False
    ),
    out_type=jax.ShapeDtypeStruct((EMBD_DIM, BATCH), jnp.float32),
    scratch_types=[
        pltpu.VMEM((B_PER_W + 16,), jnp.int32),
        pltpu.VMEM((B_PER_W, EMBD_DIM), jnp.float32),
        pltpu.VMEM((EMBD_DIM, B_PER_W), jnp.float32),
        pltpu.SemaphoreType.DMA,
        pltpu.SemaphoreType.DMA,
    ],
)
def _sc_gather(table_hbm, idx_hbm, out_hbm, idx_v, rows_v, tb_v, sem, sem2):
    wid = lax.axis_index("s") * NUM_CORES + lax.axis_index("c")
    base = wid * B_PER_W
    pltpu.sync_copy(idx_hbm.at[pl.ds(base, B_PER_W)], idx_v.at[pl.ds(0, B_PER_W)])

    def issue(b, _):
        row = idx_v[pl.ds(b, 16)][0]
        pltpu.async_copy(
            table_hbm.at[pl.ds(row, 1), :],
            rows_v.at[pl.ds(b, 1), :],
            sem,
        )
        return 0

    lax.fori_loop(0, B_PER_W, issue, 0)
    pltpu.make_async_copy(
        table_hbm.at[pl.ds(0, B_PER_W), :], rows_v, sem
    ).wait()

    lanes = lax.iota(jnp.int32, 16)
    for k in range(B_PER_W // 16):
        bvec = lax.add(lanes, jnp.int32(k * 16))

        def xpose(j, _):
            col = lax.broadcast(j, (16,))
            val = plsc.load_gather(rows_v, [bvec, col])
            tb_v[j, pl.ds(k * 16, 16)] = val
            return 0

        lax.fori_loop(0, EMBD_DIM, xpose, 0)
    pltpu.sync_copy(tb_v, out_hbm.at[:, pl.ds(base, B_PER_W)])


def kernel(spk_inds, embedding_table):
    out_t = _sc_gather(embedding_table, spk_inds.astype(jnp.int32))
    return out_t.T


# speedup vs baseline: 2.1404x; 1.0614x over previous
import functools

import jax
import jax.numpy as jnp
from jax import lax
from jax.experimental import pallas as pl
from jax.experimental.pallas import tpu as pltpu
from jax.experimental.pallas import tpu_sc as plsc

NUM_SPK = 100000
EMBD_DIM = 64
BATCH = 16384

NUM_CORES = 2
NUM_SUBCORES = 16
NW = NUM_CORES * NUM_SUBCORES
B_PER_W = BATCH // NW
IDX_CHUNK = 128
N_CHUNKS = B_PER_W // IDX_CHUNK

_mesh = plsc.VectorSubcoreMesh(core_axis_name="c", subcore_axis_name="s")


@functools.partial(
    pl.kernel,
    mesh=_mesh,
    compiler_params=pltpu.CompilerParams(
        use_tc_tiling_on_sc=True, needs_layout_passes=False
    ),
    out_type=jax.ShapeDtypeStruct((EMBD_DIM, BATCH), jnp.float32),
    scratch_types=[
        pltpu.VMEM((B_PER_W + 16,), jnp.int32),
        pltpu.VMEM((B_PER_W, EMBD_DIM), jnp.float32),
        pltpu.VMEM((EMBD_DIM, B_PER_W), jnp.float32),
        pltpu.SemaphoreType.DMA,
        pltpu.SemaphoreType.DMA,
    ],
)
def _sc_gather(table_hbm, idx_hbm, out_hbm, idx_v, rows_v, tb_v, sem, sem2):
    wid = lax.axis_index("s") * NUM_CORES + lax.axis_index("c")
    base = wid * B_PER_W
    pltpu.sync_copy(idx_hbm.at[pl.ds(base, B_PER_W)], idx_v.at[pl.ds(0, B_PER_W)])

    def issue(k, _):
        b0 = k * 16
        v = idx_v[pl.ds(b0, 16)]
        for i in range(16):
            pltpu.async_copy(
                table_hbm.at[pl.ds(v[i], 1), :],
                rows_v.at[pl.ds(b0 + i, 1), :],
                sem,
            )
        return 0

    lax.fori_loop(0, B_PER_W // 16, issue, 0)
    pltpu.make_async_copy(
        table_hbm.at[pl.ds(0, B_PER_W), :], rows_v, sem
    ).wait()

    lanes = lax.iota(jnp.int32, 16)

    def xpose(j, _):
        col = lax.broadcast(j, (16,))
        for k in range(B_PER_W // 16):
            bvec = lax.add(lanes, jnp.int32(k * 16))
            val = plsc.load_gather(rows_v, [bvec, col])
            tb_v[j, pl.ds(k * 16, 16)] = val
        return 0

    lax.fori_loop(0, EMBD_DIM, xpose, 0)
    pltpu.sync_copy(tb_v, out_hbm.at[:, pl.ds(base, B_PER_W)])


def kernel(spk_inds, embedding_table):
    out_t = _sc_gather(embedding_table, spk_inds.astype(jnp.int32))
    return out_t.T


# speedup vs baseline: 2.1632x; 1.0107x over previous
import functools

import jax
import jax.numpy as jnp
from jax import lax
from jax.experimental import pallas as pl
from jax.experimental.pallas import tpu as pltpu
from jax.experimental.pallas import tpu_sc as plsc

NUM_SPK = 100000
EMBD_DIM = 64
BATCH = 16384

NUM_CORES = 2
NUM_SUBCORES = 16
NW = NUM_CORES * NUM_SUBCORES
B_PER_W = BATCH // NW
IDX_CHUNK = 128
N_CHUNKS = B_PER_W // IDX_CHUNK

_mesh = plsc.VectorSubcoreMesh(core_axis_name="c", subcore_axis_name="s")


@functools.partial(
    pl.kernel,
    mesh=_mesh,
    compiler_params=pltpu.CompilerParams(
        use_tc_tiling_on_sc=True, needs_layout_passes=False
    ),
    out_type=jax.ShapeDtypeStruct((EMBD_DIM, BATCH), jnp.float32),
    scratch_types=[
        pltpu.VMEM((B_PER_W + 16,), jnp.int32),
        pltpu.VMEM((B_PER_W, EMBD_DIM), jnp.float32),
        pltpu.VMEM((EMBD_DIM, B_PER_W), jnp.float32),
        pltpu.SemaphoreType.DMA,
        pltpu.SemaphoreType.DMA,
    ],
)
def _sc_gather(table_hbm, idx_hbm, out_hbm, idx_v, rows_v, tb_v, sem, sem2):
    wid = lax.axis_index("s") * NUM_CORES + lax.axis_index("c")
    base = wid * B_PER_W
    pltpu.sync_copy(idx_hbm.at[pl.ds(base, B_PER_W)], idx_v.at[pl.ds(0, B_PER_W)])

    half = B_PER_W // 2

    def issue_half(h, s):
        def issue(k, _):
            b0 = h * half + k * 16
            v = idx_v[pl.ds(b0, 16)]
            for i in range(16):
                pltpu.async_copy(
                    table_hbm.at[pl.ds(v[i], 1), :],
                    rows_v.at[pl.ds(b0 + i, 1), :],
                    s,
                )
            return 0

        lax.fori_loop(0, half // 16, issue, 0)

    lanes = lax.iota(jnp.int32, 16)

    def xpose_half(h):
        def xpose(j, _):
            col = lax.broadcast(j, (16,))
            for k in range(half // 16):
                bvec = lax.add(lanes, jnp.int32(h * half + k * 16))
                val = plsc.load_gather(rows_v, [bvec, col])
                tb_v[j, pl.ds(h * half + k * 16, 16)] = val
            return 0

        lax.fori_loop(0, EMBD_DIM, xpose, 0)

    issue_half(0, sem)
    issue_half(1, sem2)
    pltpu.make_async_copy(
        table_hbm.at[pl.ds(0, half), :], rows_v.at[pl.ds(0, half), :], sem
    ).wait()
    xpose_half(0)
    pltpu.make_async_copy(
        table_hbm.at[pl.ds(0, half), :], rows_v.at[pl.ds(half, half), :], sem2
    ).wait()
    xpose_half(1)
    pltpu.sync_copy(tb_v, out_hbm.at[:, pl.ds(base, B_PER_W)])


def kernel(spk_inds, embedding_table):
    out_t = _sc_gather(embedding_table, spk_inds.astype(jnp.int32))
    return out_t.T
